# Initial kernel scaffold; baseline (speedup 1.0000x reference)
import functools

import jax
import jax.numpy as jnp
from jax import lax
from jax.experimental import pallas as pl
from jax.experimental.pallas import tpu as pltpu
from jax.experimental.pallas import tpu_sc as plsc

_SELU_ALPHA = 1.6732632423543772
_SELU_SCALE = 1.0507009873554805


@functools.partial(jax.jit, static_argnames=("with_deg",))
def _sc_aggregate(h, src, dst, *, with_deg):
    n, d = h.shape
    e = src.shape[0]
    info = plsc.get_sparse_core_info()
    nc, ns = info.num_cores, info.num_subcores
    nw = nc * ns
    chunk = 128
    per_tile = e // nw
    assert per_tile * nw == e and per_tile % 8 == 0
    full_chunks = per_tile // chunk
    rem = per_tile - full_chunks * chunk
    rows_per_tile = n // ns
    assert rows_per_tile * ns == n
    zr = 25
    assert rows_per_tile % zr == 0

    mesh = plsc.VectorSubcoreMesh(core_axis_name="c", subcore_axis_name="s")

    out_type = [jax.ShapeDtypeStruct((nc, n, d), jnp.float32)]
    if with_deg:
        out_type.append(jax.ShapeDtypeStruct((nc, n, 16), jnp.float32))

    scratch = dict(
        idx_src=pltpu.VMEM((chunk,), jnp.int32),
        idx_dst=pltpu.VMEM((chunk,), jnp.int32),
        rows=pltpu.VMEM((chunk, d), jnp.float32),
        zbuf=pltpu.VMEM((zr, d), jnp.float32),
        gsem=pltpu.SemaphoreType.DMA,
        agg_sh=pltpu.VMEM_SHARED((n, d), jnp.float32),
    )
    if rem:
        scratch.update(
            idx_src_r=pltpu.VMEM((rem,), jnp.int32),
            idx_dst_r=pltpu.VMEM((rem,), jnp.int32),
            rows_r=pltpu.VMEM((rem, d), jnp.float32),
        )
    if with_deg:
        scratch.update(
            ones=pltpu.VMEM((chunk, 16), jnp.float32),
            zbuf16=pltpu.VMEM((zr, 16), jnp.float32),
            deg_sh=pltpu.VMEM_SHARED((n, 16), jnp.float32),
        )
        if rem:
            scratch["ones_r"] = pltpu.VMEM((rem, 16), jnp.float32)

    def body(h_hbm, src_hbm, dst_hbm, *refs):
        if with_deg:
            agg_out, deg_out = refs[:2]
            refs = refs[2:]
        else:
            agg_out = refs[0]
            refs = refs[1:]
        sc = dict(zip(scratch.keys(), refs))

        cid = lax.axis_index("c")
        sid = lax.axis_index("s")
        wid = sid * nc + cid

        zvec = jnp.zeros((16,), jnp.float32)
        for r in range(zr):
            for cc in range(d // 16):
                sc["zbuf"][r, pl.ds(cc * 16, 16)] = zvec
        if with_deg:
            onev = jnp.ones((16,), jnp.float32)
            for r in range(zr):
                sc["zbuf16"][r, pl.ds(0, 16)] = zvec
            for r in range(chunk):
                sc["ones"][r, pl.ds(0, 16)] = onev
            if rem:
                for r in range(rem):
                    sc["ones_r"][r, pl.ds(0, 16)] = onev

        row0 = sid * rows_per_tile
        nz = rows_per_tile // zr

        def zero_body(j, _):
            base = row0 + j * zr
            pltpu.sync_copy(sc["zbuf"], sc["agg_sh"].at[pl.ds(base, zr)])
            if with_deg:
                pltpu.sync_copy(sc["zbuf16"], sc["deg_sh"].at[pl.ds(base, zr)])
            return 0

        lax.fori_loop(0, nz, zero_body, 0)
        plsc.subcore_barrier()

        ebase = wid * per_tile

        def chunk_body(i, _):
            b = pl.multiple_of(ebase + i * chunk, 8)
            pltpu.sync_copy(src_hbm.at[pl.ds(b, chunk)], sc["idx_src"])
            pltpu.sync_copy(dst_hbm.at[pl.ds(b, chunk)], sc["idx_dst"])
            pltpu.async_copy(h_hbm.at[sc["idx_src"]], sc["rows"], sc["gsem"]).wait()
            pltpu.sync_copy(sc["rows"], sc["agg_sh"].at[sc["idx_dst"]], add=True)
            if with_deg:
                pltpu.sync_copy(sc["ones"], sc["deg_sh"].at[sc["idx_dst"]], add=True)
            return 0

        lax.fori_loop(0, full_chunks, chunk_body, 0)

        if rem:
            b = pl.multiple_of(ebase + full_chunks * chunk, 8)
            pltpu.sync_copy(src_hbm.at[pl.ds(b, rem)], sc["idx_src_r"])
            pltpu.sync_copy(dst_hbm.at[pl.ds(b, rem)], sc["idx_dst_r"])
            pltpu.async_copy(h_hbm.at[sc["idx_src_r"]], sc["rows_r"], sc["gsem"]).wait()
            pltpu.sync_copy(sc["rows_r"], sc["agg_sh"].at[sc["idx_dst_r"]], add=True)
            if with_deg:
                pltpu.sync_copy(sc["ones_r"], sc["deg_sh"].at[sc["idx_dst_r"]], add=True)

        plsc.subcore_barrier()

        pltpu.sync_copy(
            sc["agg_sh"].at[pl.ds(row0, rows_per_tile)],
            agg_out.at[cid, pl.ds(row0, rows_per_tile)],
        )
        if with_deg:
            pltpu.sync_copy(
                sc["deg_sh"].at[pl.ds(row0, rows_per_tile)],
                deg_out.at[cid, pl.ds(row0, rows_per_tile)],
            )

    call = pl.kernel(
        body,
        out_type=out_type,
        mesh=mesh,
        scratch_types=list(scratch.values()),
    )
    return call(h, src, dst)


def _tc_layer1(x, agg, deg, w_self, w_neigh, b):
    n, d = x.shape
    h = w_self.shape[1]
    blk = 1000

    def body(x_ref, agg_ref, deg_ref, ws_ref, wn_ref, b_ref, o_ref):
        xb = x_ref[...]
        a = agg_ref[0] + agg_ref[1]
        degv = jnp.maximum(deg_ref[0, :, 0] + deg_ref[1, :, 0], 1.0)
        hn = a / degv[:, None]
        r = (
            jnp.dot(xb, ws_ref[...], preferred_element_type=jnp.float32)
            + jnp.dot(hn, wn_ref[...], preferred_element_type=jnp.float32)
            + b_ref[...]
        )
        o_ref[...] = _SELU_SCALE * jnp.where(
            r > 0, r, _SELU_ALPHA * (jnp.exp(r) - 1.0)
        )

    return pl.pallas_call(
        body,
        grid=(n // blk,),
        in_specs=[
            pl.BlockSpec((blk, d), lambda i: (i, 0)),
            pl.BlockSpec((2, blk, d), lambda i: (0, i, 0)),
            pl.BlockSpec((2, blk, 16), lambda i: (0, i, 0)),
            pl.BlockSpec((d, h), lambda i: (0, 0)),
            pl.BlockSpec((d, h), lambda i: (0, 0)),
            pl.BlockSpec((h,), lambda i: (0,)),
        ],
        out_specs=pl.BlockSpec((blk, h), lambda i: (i, 0)),
        out_shape=jax.ShapeDtypeStruct((n, h), jnp.float32),
    )(x, agg, deg, w_self, w_neigh, b)


def _tc_layer2(h1, agg, deg, w_self, w_neigh, b):
    n, h = h1.shape
    c = w_self.shape[1]
    blk = 1000

    def body(x_ref, agg_ref, deg_ref, ws_ref, wn_ref, b_ref, o_ref):
        xb = x_ref[...]
        a = agg_ref[0] + agg_ref[1]
        degv = jnp.maximum(deg_ref[0, :, 0] + deg_ref[1, :, 0], 1.0)
        hn = a / degv[:, None]
        r = (
            jnp.dot(xb, ws_ref[...], preferred_element_type=jnp.float32)
            + jnp.dot(hn, wn_ref[...], preferred_element_type=jnp.float32)
            + b_ref[...]
        )
        m = jnp.max(r, axis=1, keepdims=True)
        ex = jnp.exp(r - m)
        o_ref[...] = ex / jnp.sum(ex, axis=1, keepdims=True)

    return pl.pallas_call(
        body,
        grid=(n // blk,),
        in_specs=[
            pl.BlockSpec((blk, h), lambda i: (i, 0)),
            pl.BlockSpec((2, blk, h), lambda i: (0, i, 0)),
            pl.BlockSpec((2, blk, 16), lambda i: (0, i, 0)),
            pl.BlockSpec((h, c), lambda i: (0, 0)),
            pl.BlockSpec((h, c), lambda i: (0, 0)),
            pl.BlockSpec((c,), lambda i: (0,)),
        ],
        out_specs=pl.BlockSpec((blk, c), lambda i: (i, 0)),
        out_shape=jax.ShapeDtypeStruct((n, c), jnp.float32),
    )(h1, agg, deg, w_self, w_neigh, b)


def kernel(x, edge_index, W_self1, W_neigh1, b1, W_self2, W_neigh2, b2):
    src = edge_index[0]
    dst = edge_index[1]
    agg1, deg = _sc_aggregate(x, src, dst, with_deg=True)
    h1 = _tc_layer1(x, agg1, deg, W_self1, W_neigh1, b1)
    (agg2,) = _sc_aggregate(h1, src, dst, with_deg=False)
    return _tc_layer2(h1, agg2, deg, W_self2, W_neigh2, b2)


# speedup vs baseline: 5.5010x; 5.5010x over previous
import jax
import jax.numpy as jnp
from jax import lax
from jax.experimental import pallas as pl
from jax.experimental.pallas import tpu as pltpu
from jax.experimental.pallas import tpu_sc as plsc

_SELU_ALPHA = 1.6732632423543772
_SELU_SCALE = 1.0507009873554805

_NC, _NS = 2, 16
_CHUNK = 128


def _mesh():
    return plsc.VectorSubcoreMesh(
        core_axis_name="c", subcore_axis_name="s", num_cores=_NC, num_subcores=_NS
    )


def _row_partition(n, sid):
    blocks = n // 8
    bb = blocks // _NS
    eb = blocks % _NS
    row0 = 8 * (sid * bb + jnp.minimum(sid, eb))
    nblk = bb + jnp.where(sid < eb, 1, 0)
    return row0, nblk


def _sc_agg(h, src, dst):
    n, d = h.shape
    e = src.shape[0]
    nw = _NC * _NS
    per_tile = e // nw
    assert per_tile * nw == e and per_tile % 8 == 0 and n % 8 == 0
    full_chunks = per_tile // _CHUNK
    rem = per_tile - full_chunks * _CHUNK

    def body(h_hbm, src_hbm, dst_hbm, agg_out, idx_s, idx_d, rows, gsem,
             idx_sr, idx_dr, rows_r, agg_sh):
        cid = lax.axis_index("c")
        sid = lax.axis_index("s")
        wid = sid * _NC + cid
        row0, nblk = _row_partition(n, sid)

        zvec = jnp.zeros((16,), jnp.float32)
        for r in range(8):
            for cc in range(d // 16):
                rows[r, pl.ds(cc * 16, 16)] = zvec

        def zb(j, _):
            pltpu.sync_copy(rows.at[pl.ds(0, 8)],
                            agg_sh.at[pl.ds(row0 + j * 8, 8)])
            return 0

        lax.fori_loop(0, nblk, zb, 0)
        plsc.subcore_barrier()

        ebase = wid * per_tile

        def cb(i, _):
            b = pl.multiple_of(ebase + i * _CHUNK, 8)
            pltpu.sync_copy(src_hbm.at[pl.ds(b, _CHUNK)], idx_s)
            pltpu.sync_copy(dst_hbm.at[pl.ds(b, _CHUNK)], idx_d)
            pltpu.async_copy(h_hbm.at[idx_s], rows, gsem).wait()
            pltpu.sync_copy(rows, agg_sh.at[idx_d], add=True)
            return 0

        lax.fori_loop(0, full_chunks, cb, 0)
        if rem:
            b = pl.multiple_of(ebase + full_chunks * _CHUNK, 8)
            pltpu.sync_copy(src_hbm.at[pl.ds(b, rem)], idx_sr)
            pltpu.sync_copy(dst_hbm.at[pl.ds(b, rem)], idx_dr)
            pltpu.async_copy(h_hbm.at[idx_sr], rows_r, gsem).wait()
            pltpu.sync_copy(rows_r, agg_sh.at[idx_dr], add=True)
        plsc.subcore_barrier()

        def wb(j, _):
            base = row0 + j * 8
            pltpu.sync_copy(agg_sh.at[pl.ds(base, 8)], rows.at[pl.ds(0, 8)])
            pltpu.sync_copy(rows.at[pl.ds(0, 8)],
                            agg_out.at[cid, pl.ds(base, 8)])
            return 0

        lax.fori_loop(0, nblk, wb, 0)

    call = pl.kernel(
        body,
        out_type=[jax.ShapeDtypeStruct((_NC, n, d), jnp.float32)],
        mesh=_mesh(),
        scratch_types=[
            pltpu.VMEM((_CHUNK,), jnp.int32),
            pltpu.VMEM((_CHUNK,), jnp.int32),
            pltpu.VMEM((_CHUNK, d), jnp.float32),
            pltpu.SemaphoreType.DMA,
            pltpu.VMEM((max(rem, 8),), jnp.int32),
            pltpu.VMEM((max(rem, 8),), jnp.int32),
            pltpu.VMEM((max(rem, 8), d), jnp.float32),
            pltpu.VMEM_SHARED((n, d), jnp.float32),
        ],
    )
    (agg,) = call(h, src, dst)
    return agg


def _sc_deg(dst, n):
    e = dst.shape[0]
    d = 128
    nw = _NC * _NS
    per_tile = e // nw
    full_chunks = per_tile // _CHUNK
    rem = per_tile - full_chunks * _CHUNK

    def body(dst_hbm, deg_out, idx_d, ones, stage, idx_dr, ones_r, deg_sh):
        cid = lax.axis_index("c")
        sid = lax.axis_index("s")
        wid = sid * _NC + cid
        row0, nblk = _row_partition(n, sid)

        zvec = jnp.zeros((16,), jnp.float32)
        onev = jnp.ones((16,), jnp.float32)
        for r in range(8):
            for cc in range(d // 16):
                stage[r, pl.ds(cc * 16, 16)] = zvec
        for r in range(_CHUNK):
            for cc in range(d // 16):
                ones[r, pl.ds(cc * 16, 16)] = onev
        if rem:
            for r in range(rem):
                for cc in range(d // 16):
                    ones_r[r, pl.ds(cc * 16, 16)] = onev

        def zb(j, _):
            pltpu.sync_copy(stage, deg_sh.at[pl.ds(row0 + j * 8, 8)])
            return 0

        lax.fori_loop(0, nblk, zb, 0)
        plsc.subcore_barrier()

        ebase = wid * per_tile

        def cb(i, _):
            b = pl.multiple_of(ebase + i * _CHUNK, 8)
            pltpu.sync_copy(dst_hbm.at[pl.ds(b, _CHUNK)], idx_d)
            pltpu.sync_copy(ones, deg_sh.at[idx_d], add=True)
            return 0

        lax.fori_loop(0, full_chunks, cb, 0)
        if rem:
            b = pl.multiple_of(ebase + full_chunks * _CHUNK, 8)
            pltpu.sync_copy(dst_hbm.at[pl.ds(b, rem)], idx_dr)
            pltpu.sync_copy(ones_r, deg_sh.at[idx_dr], add=True)
        plsc.subcore_barrier()

        def wb(j, _):
            base = row0 + j * 8
            pltpu.sync_copy(deg_sh.at[pl.ds(base, 8)], stage)
            pltpu.sync_copy(stage, deg_out.at[cid, pl.ds(base, 8)])
            return 0

        lax.fori_loop(0, nblk, wb, 0)

    call = pl.kernel(
        body,
        out_type=[jax.ShapeDtypeStruct((_NC, n, d), jnp.float32)],
        mesh=_mesh(),
        scratch_types=[
            pltpu.VMEM((_CHUNK,), jnp.int32),
            pltpu.VMEM((_CHUNK, d), jnp.float32),
            pltpu.VMEM((8, d), jnp.float32),
            pltpu.VMEM((max(rem, 8),), jnp.int32),
            pltpu.VMEM((max(rem, 8), d), jnp.float32),
            pltpu.VMEM_SHARED((n, d), jnp.float32),
        ],
    )
    (deg,) = call(dst)
    return deg


def _tc_layer1(x, agg, deg, w_self, w_neigh, b):
    n, d = x.shape
    h = w_self.shape[1]
    blk = 1000

    def body(x_ref, agg_ref, deg_ref, ws_ref, wn_ref, b_ref, o_ref):
        xb = x_ref[...]
        a = agg_ref[0] + agg_ref[1]
        degv = jnp.maximum(deg_ref[0, :, 0] + deg_ref[1, :, 0], 1.0)
        hn = a / degv[:, None]
        r = (
            jnp.dot(xb, ws_ref[...], preferred_element_type=jnp.float32)
            + jnp.dot(hn, wn_ref[...], preferred_element_type=jnp.float32)
            + b_ref[...]
        )
        o_ref[...] = _SELU_SCALE * jnp.where(
            r > 0, r, _SELU_ALPHA * (jnp.exp(r) - 1.0)
        )

    return pl.pallas_call(
        body,
        grid=(n // blk,),
        in_specs=[
            pl.BlockSpec((blk, d), lambda i: (i, 0)),
            pl.BlockSpec((2, blk, d), lambda i: (0, i, 0)),
            pl.BlockSpec((2, blk, 128), lambda i: (0, i, 0)),
            pl.BlockSpec((d, h), lambda i: (0, 0)),
            pl.BlockSpec((d, h), lambda i: (0, 0)),
            pl.BlockSpec((h,), lambda i: (0,)),
        ],
        out_specs=pl.BlockSpec((blk, h), lambda i: (i, 0)),
        out_shape=jax.ShapeDtypeStruct((n, h), jnp.float32),
    )(x, agg, deg, w_self, w_neigh, b)


def _tc_layer2(h1, agg, deg, w_self, w_neigh, b):
    n, h = h1.shape
    c = w_self.shape[1]
    blk = 1000

    def body(x_ref, agg_ref, deg_ref, ws_ref, wn_ref, b_ref, o_ref):
        xb = x_ref[...]
        a = agg_ref[0] + agg_ref[1]
        degv = jnp.maximum(deg_ref[0, :, 0] + deg_ref[1, :, 0], 1.0)
        hn = a / degv[:, None]
        r = (
            jnp.dot(xb, ws_ref[...], preferred_element_type=jnp.float32)
            + jnp.dot(hn, wn_ref[...], preferred_element_type=jnp.float32)
            + b_ref[...]
        )
        m = jnp.max(r, axis=1, keepdims=True)
        ex = jnp.exp(r - m)
        o_ref[...] = ex / jnp.sum(ex, axis=1, keepdims=True)

    return pl.pallas_call(
        body,
        grid=(n // blk,),
        in_specs=[
            pl.BlockSpec((blk, h), lambda i: (i, 0)),
            pl.BlockSpec((2, blk, h), lambda i: (0, i, 0)),
            pl.BlockSpec((2, blk, 128), lambda i: (0, i, 0)),
            pl.BlockSpec((h, c), lambda i: (0, 0)),
            pl.BlockSpec((h, c), lambda i: (0, 0)),
            pl.BlockSpec((c,), lambda i: (0,)),
        ],
        out_specs=pl.BlockSpec((blk, c), lambda i: (i, 0)),
        out_shape=jax.ShapeDtypeStruct((n, c), jnp.float32),
    )(h1, agg, deg, w_self, w_neigh, b)


def kernel(x, edge_index, W_self1, W_neigh1, b1, W_self2, W_neigh2, b2):
    n = x.shape[0]
    src = edge_index[0]
    dst = edge_index[1]
    deg = _sc_deg(dst, n)
    agg1 = _sc_agg(x, src, dst)
    h1 = _tc_layer1(x, agg1, deg, W_self1, W_neigh1, b1)
    agg2 = _sc_agg(h1, src, dst)
    return _tc_layer2(h1, agg2, deg, W_self2, W_neigh2, b2)


# speedup vs baseline: 8.6033x; 1.5640x over previous
import jax
import jax.numpy as jnp
from jax import lax
from jax.experimental import pallas as pl
from jax.experimental.pallas import tpu as pltpu
from jax.experimental.pallas import tpu_sc as plsc

_SELU_ALPHA = 1.6732632423543772
_SELU_SCALE = 1.0507009873554805

_NC, _NS = 2, 16
_CHUNK = 128


def _mesh():
    return plsc.VectorSubcoreMesh(
        core_axis_name="c", subcore_axis_name="s", num_cores=_NC, num_subcores=_NS
    )


def _row_partition(n, sid):
    blocks = n // 8
    bb = blocks // _NS
    eb = blocks % _NS
    row0 = 8 * (sid * bb + jnp.minimum(sid, eb))
    nblk = bb + jnp.where(sid < eb, 1, 0)
    return row0, nblk


def _sc_agg(h, src, dst):
    n, d = h.shape
    e = src.shape[0]
    nw = _NC * _NS
    per_tile = e // nw
    assert per_tile * nw == e and per_tile % 8 == 0 and n % 8 == 0
    full_chunks = per_tile // _CHUNK
    rem = per_tile - full_chunks * _CHUNK

    pipe_main = max(full_chunks - 2, 0)

    def body(h_hbm, src_hbm, dst_hbm, agg_out, idx_s, idx_d, rows, sem0, sem1,
             idx_sr, idx_dr, rows_r, agg_sh):
        sems = (sem0, sem1)
        cid = lax.axis_index("c")
        sid = lax.axis_index("s")
        wid = sid * _NC + cid
        row0, nblk = _row_partition(n, sid)

        zvec = jnp.zeros((16,), jnp.float32)
        for r in range(_CHUNK):
            for cc in range(d // 16):
                rows[0, r, pl.ds(cc * 16, 16)] = zvec

        nzh = nblk // 16

        def zbh(j, _):
            pltpu.sync_copy(rows.at[0],
                            agg_sh.at[pl.ds(row0 + j * _CHUNK, _CHUNK)])
            return 0

        def zbt(j, _):
            pltpu.sync_copy(rows.at[0, pl.ds(0, 8)],
                            agg_sh.at[pl.ds(row0 + nzh * _CHUNK + j * 8, 8)])
            return 0

        lax.fori_loop(0, nzh, zbh, 0)
        lax.fori_loop(0, nblk - nzh * 16, zbt, 0)
        plsc.subcore_barrier()

        ebase = wid * per_tile

        def load_and_fire(c, p):
            b = pl.multiple_of(ebase + c * _CHUNK, 8)
            pltpu.sync_copy(src_hbm.at[pl.ds(b, _CHUNK)], idx_s.at[p])
            pltpu.sync_copy(dst_hbm.at[pl.ds(b, _CHUNK)], idx_d.at[p])
            pltpu.async_copy(h_hbm.at[idx_s.at[p]], rows.at[p], sems[p])

        def drain_and_scatter(p):
            pltpu.make_async_copy(
                h_hbm.at[idx_s.at[p]], rows.at[p], sems[p]).wait()
            pltpu.sync_copy(rows.at[p], agg_sh.at[idx_d.at[p]], add=True)

        for p in range(2):
            load_and_fire(p, p)

        def cb(i2, _):
            for p in range(2):
                drain_and_scatter(p)
                load_and_fire(2 * i2 + p + 2, p)
            return 0

        lax.fori_loop(0, pipe_main // 2, cb, 0)
        for p in range(full_chunks - pipe_main):
            drain_and_scatter(p)

        if rem:
            b = pl.multiple_of(ebase + full_chunks * _CHUNK, 8)
            pltpu.sync_copy(src_hbm.at[pl.ds(b, rem)], idx_sr)
            pltpu.sync_copy(dst_hbm.at[pl.ds(b, rem)], idx_dr)
            pltpu.async_copy(h_hbm.at[idx_sr], rows_r, sem0).wait()
            pltpu.sync_copy(rows_r, agg_sh.at[idx_dr], add=True)
        plsc.subcore_barrier()

        def wbh(j, _):
            base = row0 + j * _CHUNK
            pltpu.sync_copy(agg_sh.at[pl.ds(base, _CHUNK)], rows.at[0])
            pltpu.sync_copy(rows.at[0], agg_out.at[cid, pl.ds(base, _CHUNK)])
            return 0

        def wbt(j, _):
            base = row0 + nzh * _CHUNK + j * 8
            pltpu.sync_copy(agg_sh.at[pl.ds(base, 8)], rows.at[0, pl.ds(0, 8)])
            pltpu.sync_copy(rows.at[0, pl.ds(0, 8)],
                            agg_out.at[cid, pl.ds(base, 8)])
            return 0

        lax.fori_loop(0, nzh, wbh, 0)
        lax.fori_loop(0, nblk - nzh * 16, wbt, 0)

    call = pl.kernel(
        body,
        out_type=[jax.ShapeDtypeStruct((_NC, n, d), jnp.float32)],
        mesh=_mesh(),
        scratch_types=[
            pltpu.VMEM((2, _CHUNK), jnp.int32),
            pltpu.VMEM((2, _CHUNK), jnp.int32),
            pltpu.VMEM((2, _CHUNK, d), jnp.float32),
            pltpu.SemaphoreType.DMA,
            pltpu.SemaphoreType.DMA,
            pltpu.VMEM((max(rem, 8),), jnp.int32),
            pltpu.VMEM((max(rem, 8),), jnp.int32),
            pltpu.VMEM((max(rem, 8), d), jnp.float32),
            pltpu.VMEM_SHARED((n, d), jnp.float32),
        ],
    )
    (agg,) = call(h, src, dst)
    return agg


def _sc_deg(dst, n):
    e = dst.shape[0]
    d = 128
    nw = _NC * _NS
    per_tile = e // nw
    full_chunks = per_tile // _CHUNK
    rem = per_tile - full_chunks * _CHUNK

    pipe_main = max(full_chunks - 2, 0)

    def body(dst_hbm, deg_out, idx_d, ones, stage, sem0, sem1, idx_dr, ones_r,
             deg_sh):
        sems = (sem0, sem1)
        cid = lax.axis_index("c")
        sid = lax.axis_index("s")
        wid = sid * _NC + cid
        row0, nblk = _row_partition(n, sid)

        zvec = jnp.zeros((16,), jnp.float32)
        onev = jnp.ones((16,), jnp.float32)
        for r in range(8):
            for cc in range(d // 16):
                stage[r, pl.ds(cc * 16, 16)] = zvec
        for r in range(_CHUNK):
            for cc in range(d // 16):
                ones[r, pl.ds(cc * 16, 16)] = onev
        if rem:
            for r in range(rem):
                for cc in range(d // 16):
                    ones_r[r, pl.ds(cc * 16, 16)] = onev

        def zb(j, _):
            pltpu.sync_copy(stage, deg_sh.at[pl.ds(row0 + j * 8, 8)])
            return 0

        lax.fori_loop(0, nblk, zb, 0)
        plsc.subcore_barrier()

        ebase = wid * per_tile

        def fire_idx(c, p):
            b = pl.multiple_of(ebase + c * _CHUNK, 8)
            pltpu.async_copy(dst_hbm.at[pl.ds(b, _CHUNK)], idx_d.at[p], sems[p])

        def drain_and_scatter(c, p):
            b = pl.multiple_of(ebase + c * _CHUNK, 8)
            pltpu.make_async_copy(
                dst_hbm.at[pl.ds(b, _CHUNK)], idx_d.at[p], sems[p]).wait()
            pltpu.sync_copy(ones, deg_sh.at[idx_d.at[p]], add=True)

        for p in range(2):
            fire_idx(p, p)

        def cb(i2, _):
            for p in range(2):
                drain_and_scatter(2 * i2 + p, p)
                fire_idx(2 * i2 + p + 2, p)
            return 0

        lax.fori_loop(0, pipe_main // 2, cb, 0)
        for p in range(full_chunks - pipe_main):
            drain_and_scatter(pipe_main + p, p)

        if rem:
            b = pl.multiple_of(ebase + full_chunks * _CHUNK, 8)
            pltpu.async_copy(dst_hbm.at[pl.ds(b, rem)], idx_dr, sem0).wait()
            pltpu.sync_copy(ones_r, deg_sh.at[idx_dr], add=True)
        plsc.subcore_barrier()

        nzh = nblk // 16

        def wbh(j, _):
            base = row0 + j * _CHUNK
            pltpu.sync_copy(deg_sh.at[pl.ds(base, _CHUNK)], ones)
            pltpu.sync_copy(ones, deg_out.at[cid, pl.ds(base, _CHUNK)])
            return 0

        def wbt(j, _):
            base = row0 + nzh * _CHUNK + j * 8
            pltpu.sync_copy(deg_sh.at[pl.ds(base, 8)], stage)
            pltpu.sync_copy(stage, deg_out.at[cid, pl.ds(base, 8)])
            return 0

        lax.fori_loop(0, nzh, wbh, 0)
        lax.fori_loop(0, nblk - nzh * 16, wbt, 0)

    call = pl.kernel(
        body,
        out_type=[jax.ShapeDtypeStruct((_NC, n, d), jnp.float32)],
        mesh=_mesh(),
        scratch_types=[
            pltpu.VMEM((2, _CHUNK), jnp.int32),
            pltpu.VMEM((_CHUNK, d), jnp.float32),
            pltpu.VMEM((8, d), jnp.float32),
            pltpu.SemaphoreType.DMA,
            pltpu.SemaphoreType.DMA,
            pltpu.VMEM((max(rem, 8),), jnp.int32),
            pltpu.VMEM((max(rem, 8), d), jnp.float32),
            pltpu.VMEM_SHARED((n, d), jnp.float32),
        ],
    )
    (deg,) = call(dst)
    return deg


def _tc_layer1(x, agg, deg, w_self, w_neigh, b):
    n, d = x.shape
    h = w_self.shape[1]
    blk = 1000

    def body(x_ref, agg_ref, deg_ref, ws_ref, wn_ref, b_ref, o_ref):
        xb = x_ref[...]
        a = agg_ref[0] + agg_ref[1]
        degv = jnp.maximum(deg_ref[0, :, 0] + deg_ref[1, :, 0], 1.0)
        hn = a / degv[:, None]
        r = (
            jnp.dot(xb, ws_ref[...], preferred_element_type=jnp.float32)
            + jnp.dot(hn, wn_ref[...], preferred_element_type=jnp.float32)
            + b_ref[...]
        )
        o_ref[...] = _SELU_SCALE * jnp.where(
            r > 0, r, _SELU_ALPHA * (jnp.exp(r) - 1.0)
        )

    return pl.pallas_call(
        body,
        grid=(n // blk,),
        in_specs=[
            pl.BlockSpec((blk, d), lambda i: (i, 0)),
            pl.BlockSpec((2, blk, d), lambda i: (0, i, 0)),
            pl.BlockSpec((2, blk, 128), lambda i: (0, i, 0)),
            pl.BlockSpec((d, h), lambda i: (0, 0)),
            pl.BlockSpec((d, h), lambda i: (0, 0)),
            pl.BlockSpec((h,), lambda i: (0,)),
        ],
        out_specs=pl.BlockSpec((blk, h), lambda i: (i, 0)),
        out_shape=jax.ShapeDtypeStruct((n, h), jnp.float32),
    )(x, agg, deg, w_self, w_neigh, b)


def _tc_layer2(h1, agg, deg, w_self, w_neigh, b):
    n, h = h1.shape
    c = w_self.shape[1]
    blk = 1000

    def body(x_ref, agg_ref, deg_ref, ws_ref, wn_ref, b_ref, o_ref):
        xb = x_ref[...]
        a = agg_ref[0] + agg_ref[1]
        degv = jnp.maximum(deg_ref[0, :, 0] + deg_ref[1, :, 0], 1.0)
        hn = a / degv[:, None]
        r = (
            jnp.dot(xb, ws_ref[...], preferred_element_type=jnp.float32)
            + jnp.dot(hn, wn_ref[...], preferred_element_type=jnp.float32)
            + b_ref[...]
        )
        m = jnp.max(r, axis=1, keepdims=True)
        ex = jnp.exp(r - m)
        o_ref[...] = ex / jnp.sum(ex, axis=1, keepdims=True)

    return pl.pallas_call(
        body,
        grid=(n // blk,),
        in_specs=[
            pl.BlockSpec((blk, h), lambda i: (i, 0)),
            pl.BlockSpec((2, blk, h), lambda i: (0, i, 0)),
            pl.BlockSpec((2, blk, 128), lambda i: (0, i, 0)),
            pl.BlockSpec((h, c), lambda i: (0, 0)),
            pl.BlockSpec((h, c), lambda i: (0, 0)),
            pl.BlockSpec((c,), lambda i: (0,)),
        ],
        out_specs=pl.BlockSpec((blk, c), lambda i: (i, 0)),
        out_shape=jax.ShapeDtypeStruct((n, c), jnp.float32),
    )(h1, agg, deg, w_self, w_neigh, b)


def kernel(x, edge_index, W_self1, W_neigh1, b1, W_self2, W_neigh2, b2):
    n = x.shape[0]
    src = edge_index[0]
    dst = edge_index[1]
    deg = _sc_deg(dst, n)
    agg1 = _sc_agg(x, src, dst)
    h1 = _tc_layer1(x, agg1, deg, W_self1, W_neigh1, b1)
    agg2 = _sc_agg(h1, src, dst)
    return _tc_layer2(h1, agg2, deg, W_self2, W_neigh2, b2)
